# baseline (speedup 1.0000x reference)
import jax
import jax.numpy as jnp
from jax import lax
from jax.experimental import pallas as pl
from jax.experimental.pallas import tpu as pltpu
from jax.experimental.pallas import tpu_sc as plsc

V = 100000
F = 26
D = 16
NUM_FEATURES = 13
B = 16384
H1 = 128
H2 = 64

N = B * F
NC = 2
NS = 16
NW = NC * NS
PER_W = N // NW
COLS = B // NW
CHUNK = 128
NCH = PER_W // CHUNK
K = 8
L = 16
FP = 32


def _sc_gather_body(tbl_hbm, idxT_hbm, fv_hbm, bv_hbm, out_hbm,
                    cat_v, fv_v, bv_v, idx_v, buf, sem):
    c = lax.axis_index("c")
    s = lax.axis_index("s")
    wid = s * NC + c
    col0 = wid * COLS
    pltpu.sync_copy(idxT_hbm.at[:, pl.ds(col0, COLS)], cat_v)
    pltpu.sync_copy(fv_hbm, fv_v)
    pltpu.sync_copy(bv_hbm, bv_v)

    def build(j, carry):
        for g in range(CHUNK // L):
            o = j * CHUNK + g * L
            fv = fv_v[pl.ds(o, L)]
            bv = bv_v[pl.ds(o, L)]
            raw = plsc.load_gather(cat_v, [fv, bv])
            idx_v[pl.ds(o, L)] = raw + fv * V
        return carry

    lax.fori_loop(0, NCH, build, 0)

    def group(t, carry):
        j0 = t * K
        for b in range(K):
            pltpu.async_copy(
                tbl_hbm.at[idx_v.at[pl.ds((j0 + b) * CHUNK, CHUNK)]],
                buf.at[pl.ds(b * CHUNK, CHUNK)],
                sem,
            )
        for b in range(K):
            pltpu.make_async_copy(
                tbl_hbm.at[idx_v.at[pl.ds((j0 + b) * CHUNK, CHUNK)]],
                buf.at[pl.ds(b * CHUNK, CHUNK)],
                sem,
            ).wait()
        pltpu.sync_copy(
            buf, out_hbm.at[pl.ds((wid * NCH + j0) * CHUNK, K * CHUNK)]
        )
        return carry

    lax.fori_loop(0, NCH // K, group, 0)


@jax.jit
def _sc_gather(tables, idxT):
    pos = jnp.arange(PER_W, dtype=jnp.int32)
    fv = pos % F
    bv = pos // F
    mesh = plsc.VectorSubcoreMesh(core_axis_name="c", subcore_axis_name="s")
    return pl.kernel(
        _sc_gather_body,
        out_type=jax.ShapeDtypeStruct((N, D), jnp.float32),
        mesh=mesh,
        compiler_params=pltpu.CompilerParams(use_tc_tiling_on_sc=False, needs_layout_passes=False),
        scratch_types=[
            pltpu.VMEM((FP, COLS), jnp.int32),
            pltpu.VMEM((PER_W,), jnp.int32),
            pltpu.VMEM((PER_W,), jnp.int32),
            pltpu.VMEM((PER_W,), jnp.int32),
            pltpu.VMEM((K * CHUNK, D), jnp.float32),
            pltpu.SemaphoreType.DMA,
        ],
    )(tables, idxT, fv, bv)


BB = 1024


def _mlp_body(xn_ref, emb_ref, w1a_ref, w1b_ref, b1_ref, g1_ref, be1_ref,
              w2_ref, b2_ref, g2_ref, be2_ref, w3_ref, b3_ref, out_ref):
    hp = jax.lax.Precision.HIGHEST
    x1 = (
        jnp.dot(xn_ref[...], w1a_ref[...], precision=hp,
                preferred_element_type=jnp.float32)
        + jnp.dot(emb_ref[...], w1b_ref[...], precision=hp,
                  preferred_element_type=jnp.float32)
        + b1_ref[...]
    )
    m1 = jnp.mean(x1, axis=-1, keepdims=True)
    v1 = jnp.mean((x1 - m1) * (x1 - m1), axis=-1, keepdims=True)
    h1 = (x1 - m1) / jnp.sqrt(v1 + 1e-5) * g1_ref[...] + be1_ref[...]
    h1 = jnp.maximum(h1, 0.0)

    x2 = jnp.dot(h1, w2_ref[...], precision=hp,
                 preferred_element_type=jnp.float32) + b2_ref[...]
    m2 = jnp.mean(x2, axis=-1, keepdims=True)
    v2 = jnp.mean((x2 - m2) * (x2 - m2), axis=-1, keepdims=True)
    h2 = (x2 - m2) / jnp.sqrt(v2 + 1e-5) * g2_ref[...] + be2_ref[...]
    h2 = jnp.maximum(h2, 0.0)

    out_ref[...] = jnp.dot(h2, w3_ref[...], precision=hp,
                           preferred_element_type=jnp.float32) + b3_ref[...]


@jax.jit
def _mlp(x_num, emb, W1a, W1b, b1, g1, be1, W2, b2, g2, be2, W3, b3):
    full = lambda shape: pl.BlockSpec(shape, lambda i: (0, 0))
    return pl.pallas_call(
        _mlp_body,
        grid=(B // BB,),
        in_specs=[
            pl.BlockSpec((BB, NUM_FEATURES), lambda i: (i, 0)),
            pl.BlockSpec((BB, F * D), lambda i: (i, 0)),
            full((NUM_FEATURES, H1)),
            full((F * D, H1)),
            full((1, H1)),
            full((1, H1)),
            full((1, H1)),
            full((H1, H2)),
            full((1, H2)),
            full((1, H2)),
            full((1, H2)),
            full((H2, 1)),
            full((1, 1)),
        ],
        out_specs=pl.BlockSpec((BB, 1), lambda i: (i, 0)),
        out_shape=jax.ShapeDtypeStruct((B, 1), jnp.float32),
    )(x_num, emb, W1a, W1b, b1.reshape(1, H1), g1.reshape(1, H1),
      be1.reshape(1, H1), W2, b2.reshape(1, H2), g2.reshape(1, H2),
      be2.reshape(1, H2), W3, b3.reshape(1, 1))


def kernel(x_num, x_cat, tables, W1, b1, g1, be1, W2, b2, g2, be2, W3, b3):
    idxT = jnp.pad(x_cat.T, ((0, FP - F), (0, 0)))
    emb = _sc_gather(tables, idxT).reshape(B, F * D)
    W1a = W1[:NUM_FEATURES]
    W1b = W1[NUM_FEATURES:]
    return _mlp(x_num, emb, W1a, W1b, b1, g1, be1, W2, b2, g2, be2, W3, b3)


# speedup vs baseline: 1.0001x; 1.0001x over previous
import jax
import jax.numpy as jnp
from jax import lax
from jax.experimental import pallas as pl
from jax.experimental.pallas import tpu as pltpu
from jax.experimental.pallas import tpu_sc as plsc

V = 100000
F = 26
D = 16
NUM_FEATURES = 13
B = 16384
H1 = 128
H2 = 64

N = B * F
NC = 2
NS = 16
NW = NC * NS
PER_W = N // NW
COLS = B // NW
CHUNK = 128
NCH = PER_W // CHUNK
K = 8
L = 16
FP = 32


def _sc_gather_body(tbl_hbm, idxU_hbm, vrow_hbm, vcol_hbm, foff_hbm, out_hbm,
                    cat_v, vrow_v, vcol_v, foff_v, idx_v, buf, sem):
    c = lax.axis_index("c")
    s = lax.axis_index("s")
    wid = s * NC + c
    c0 = wid * (COLS // 128)
    for m in range(FP // 8):
        pltpu.sync_copy(
            idxU_hbm.at[pl.ds((m * (B // 128) + c0) * 8, (COLS // 128) * 8)],
            cat_v.at[pl.ds(m * (COLS // 128) * 8, (COLS // 128) * 8)],
        )
    pltpu.sync_copy(vrow_hbm, vrow_v)
    pltpu.sync_copy(vcol_hbm, vcol_v)
    pltpu.sync_copy(foff_hbm, foff_v)

    def build(j, carry):
        for g in range(CHUNK // L):
            o = j * CHUNK + g * L
            rv = vrow_v[pl.ds(o, L)]
            cv = vcol_v[pl.ds(o, L)]
            raw = plsc.load_gather(cat_v, [rv, cv])
            idx_v[pl.ds(o, L)] = raw + foff_v[pl.ds(o, L)]
        return carry

    lax.fori_loop(0, NCH, build, 0)

    def group(t, carry):
        j0 = t * K
        for b in range(K):
            pltpu.async_copy(
                tbl_hbm.at[idx_v.at[pl.ds((j0 + b) * CHUNK, CHUNK)]],
                buf.at[pl.ds(b * CHUNK, CHUNK)],
                sem,
            )
        for b in range(K):
            pltpu.make_async_copy(
                tbl_hbm.at[idx_v.at[pl.ds((j0 + b) * CHUNK, CHUNK)]],
                buf.at[pl.ds(b * CHUNK, CHUNK)],
                sem,
            ).wait()
        pltpu.sync_copy(
            buf, out_hbm.at[pl.ds((wid * NCH + j0) * CHUNK, K * CHUNK)]
        )
        return carry

    lax.fori_loop(0, NCH // K, group, 0)


@jax.jit
def _sc_gather(tables, idxU):
    pos = jnp.arange(PER_W, dtype=jnp.int32)
    f = pos % F
    j = pos // F
    vrow = (f // 8) * ((COLS // 128) * 8) + (j // 128) * 8 + f % 8
    vcol = j % 128
    foff = f * V
    mesh = plsc.VectorSubcoreMesh(core_axis_name="c", subcore_axis_name="s")
    return pl.kernel(
        _sc_gather_body,
        out_type=jax.ShapeDtypeStruct((N, D), jnp.float32),
        mesh=mesh,
        compiler_params=pltpu.CompilerParams(use_tc_tiling_on_sc=False, needs_layout_passes=False),
        scratch_types=[
            pltpu.VMEM((FP * COLS // 128, 128), jnp.int32),
            pltpu.VMEM((PER_W,), jnp.int32),
            pltpu.VMEM((PER_W,), jnp.int32),
            pltpu.VMEM((PER_W,), jnp.int32),
            pltpu.VMEM((PER_W,), jnp.int32),
            pltpu.VMEM((K * CHUNK, D), jnp.float32),
            pltpu.SemaphoreType.DMA,
        ],
    )(tables, idxU, vrow, vcol, foff)


BB = 1024


def _mlp_body(xn_ref, emb_ref, w1a_ref, w1b_ref, b1_ref, g1_ref, be1_ref,
              w2_ref, b2_ref, g2_ref, be2_ref, w3_ref, b3_ref, out_ref):
    hp = jax.lax.Precision.HIGHEST
    x1 = (
        jnp.dot(xn_ref[...], w1a_ref[...], precision=hp,
                preferred_element_type=jnp.float32)
        + jnp.dot(emb_ref[...], w1b_ref[...], precision=hp,
                  preferred_element_type=jnp.float32)
        + b1_ref[...]
    )
    m1 = jnp.mean(x1, axis=-1, keepdims=True)
    v1 = jnp.mean((x1 - m1) * (x1 - m1), axis=-1, keepdims=True)
    h1 = (x1 - m1) / jnp.sqrt(v1 + 1e-5) * g1_ref[...] + be1_ref[...]
    h1 = jnp.maximum(h1, 0.0)

    x2 = jnp.dot(h1, w2_ref[...], precision=hp,
                 preferred_element_type=jnp.float32) + b2_ref[...]
    m2 = jnp.mean(x2, axis=-1, keepdims=True)
    v2 = jnp.mean((x2 - m2) * (x2 - m2), axis=-1, keepdims=True)
    h2 = (x2 - m2) / jnp.sqrt(v2 + 1e-5) * g2_ref[...] + be2_ref[...]
    h2 = jnp.maximum(h2, 0.0)

    out_ref[...] = jnp.dot(h2, w3_ref[...], precision=hp,
                           preferred_element_type=jnp.float32) + b3_ref[...]


@jax.jit
def _mlp(x_num, emb, W1a, W1b, b1, g1, be1, W2, b2, g2, be2, W3, b3):
    full = lambda shape: pl.BlockSpec(shape, lambda i: (0, 0))
    return pl.pallas_call(
        _mlp_body,
        grid=(B // BB,),
        in_specs=[
            pl.BlockSpec((BB, NUM_FEATURES), lambda i: (i, 0)),
            pl.BlockSpec((BB, F * D), lambda i: (i, 0)),
            full((NUM_FEATURES, H1)),
            full((F * D, H1)),
            full((1, H1)),
            full((1, H1)),
            full((1, H1)),
            full((H1, H2)),
            full((1, H2)),
            full((1, H2)),
            full((1, H2)),
            full((H2, 1)),
            full((1, 1)),
        ],
        out_specs=pl.BlockSpec((BB, 1), lambda i: (i, 0)),
        out_shape=jax.ShapeDtypeStruct((B, 1), jnp.float32),
    )(x_num, emb, W1a, W1b, b1.reshape(1, H1), g1.reshape(1, H1),
      be1.reshape(1, H1), W2, b2.reshape(1, H2), g2.reshape(1, H2),
      be2.reshape(1, H2), W3, b3.reshape(1, 1))


def kernel(x_num, x_cat, tables, W1, b1, g1, be1, W2, b2, g2, be2, W3, b3):
    idxU = (
        jnp.pad(x_cat.T, ((0, FP - F), (0, 0)))
        .reshape(FP // 8, 8, B // 128, 128)
        .transpose(0, 2, 1, 3)
        .reshape(FP // 8 * (B // 128) * 8, 128)
    )
    emb = _sc_gather(tables, idxU).reshape(B, F * D)
    W1a = W1[:NUM_FEATURES]
    W1b = W1[NUM_FEATURES:]
    return _mlp(x_num, emb, W1a, W1b, b1, g1, be1, W2, b2, g2, be2, W3, b3)


# speedup vs baseline: 1.0101x; 1.0100x over previous
import jax
import jax.numpy as jnp
from jax import lax
from jax.experimental import pallas as pl
from jax.experimental.pallas import tpu as pltpu
from jax.experimental.pallas import tpu_sc as plsc

V = 100000
F = 26
D = 16
NUM_FEATURES = 13
B = 16384
H1 = 128
H2 = 64

N = B * F
NC = 2
NS = 16
NW = NC * NS
PER_W = N // NW
COLS = B // NW
CHUNK = 128
NCH = PER_W // CHUNK
K = 8
L = 16
FP = 32


def _sc_idx_body(idxU_hbm, vrow_hbm, vcol_hbm, foff_hbm, out_hbm,
                 cat_v, vrow_v, vcol_v, foff_v, idx_v):
    c = lax.axis_index("c")
    s = lax.axis_index("s")
    wid = s * NC + c
    c0 = wid * (COLS // 128)
    for m in range(FP // 8):
        pltpu.sync_copy(
            idxU_hbm.at[pl.ds((m * (B // 128) + c0) * 8, (COLS // 128) * 8)],
            cat_v.at[pl.ds(m * (COLS // 128) * 8, (COLS // 128) * 8)],
        )
    pltpu.sync_copy(vrow_hbm, vrow_v)
    pltpu.sync_copy(vcol_hbm, vcol_v)
    pltpu.sync_copy(foff_hbm, foff_v)

    def build(j, carry):
        for g in range(CHUNK // L):
            o = j * CHUNK + g * L
            rv = vrow_v[pl.ds(o, L)]
            cv = vcol_v[pl.ds(o, L)]
            raw = plsc.load_gather(cat_v, [rv, cv])
            idx_v[pl.ds(o, L)] = raw + foff_v[pl.ds(o, L)]
        return carry

    lax.fori_loop(0, NCH, build, 0)
    pltpu.sync_copy(idx_v, out_hbm.at[pl.ds(wid * PER_W, PER_W)])


@jax.jit
def _sc_idx(idxU):
    pos = jnp.arange(PER_W, dtype=jnp.int32)
    f = pos % F
    j = pos // F
    vrow = (f // 8) * ((COLS // 128) * 8) + (j // 128) * 8 + f % 8
    vcol = j % 128
    foff = f * V
    mesh = plsc.VectorSubcoreMesh(core_axis_name="c", subcore_axis_name="s")
    return pl.kernel(
        _sc_idx_body,
        out_type=jax.ShapeDtypeStruct((N,), jnp.int32),
        mesh=mesh,
        compiler_params=pltpu.CompilerParams(
            use_tc_tiling_on_sc=False, needs_layout_passes=False),
        scratch_types=[
            pltpu.VMEM((FP * COLS // 128, 128), jnp.int32),
            pltpu.VMEM((PER_W,), jnp.int32),
            pltpu.VMEM((PER_W,), jnp.int32),
            pltpu.VMEM((PER_W,), jnp.int32),
            pltpu.VMEM((PER_W,), jnp.int32),
        ],
    )(idxU, vrow, vcol, foff)


def _sc_gather_body(tbl_hbm, idx_hbm, out_hbm, idx_v, buf, sem):
    c = lax.axis_index("c")
    s = lax.axis_index("s")
    wid = s * NC + c
    row0 = wid * NCH
    pltpu.sync_copy(idx_hbm.at[pl.ds(row0, NCH)], idx_v)

    def group(t, carry):
        j0 = t * K
        for b in range(K):
            pltpu.async_copy(
                tbl_hbm.at[idx_v.at[j0 + b]],
                buf.at[pl.ds(b * CHUNK, CHUNK)],
                sem,
            )
        for b in range(K):
            pltpu.make_async_copy(
                tbl_hbm.at[idx_v.at[j0 + b]],
                buf.at[pl.ds(b * CHUNK, CHUNK)],
                sem,
            ).wait()
        pltpu.sync_copy(
            buf, out_hbm.at[pl.ds((row0 + j0) * CHUNK, K * CHUNK)]
        )
        return carry

    lax.fori_loop(0, NCH // K, group, 0)


@jax.jit
def _sc_gather(tables, idx_rows):
    mesh = plsc.VectorSubcoreMesh(core_axis_name="c", subcore_axis_name="s")
    return pl.kernel(
        _sc_gather_body,
        out_type=jax.ShapeDtypeStruct((N, D), jnp.float32),
        mesh=mesh,
        compiler_params=pltpu.CompilerParams(use_tc_tiling_on_sc=False),
        scratch_types=[
            pltpu.VMEM((NCH, CHUNK), jnp.int32),
            pltpu.VMEM((K * CHUNK, D), jnp.float32),
            pltpu.SemaphoreType.DMA,
        ],
    )(tables, idx_rows)


BB = 1024


def _mlp_body(xn_ref, emb_ref, w1a_ref, w1b_ref, b1_ref, g1_ref, be1_ref,
              w2_ref, b2_ref, g2_ref, be2_ref, w3_ref, b3_ref, out_ref):
    hp = jax.lax.Precision.HIGHEST
    x1 = (
        jnp.dot(xn_ref[...], w1a_ref[...], precision=hp,
                preferred_element_type=jnp.float32)
        + jnp.dot(emb_ref[...], w1b_ref[...], precision=hp,
                  preferred_element_type=jnp.float32)
        + b1_ref[...]
    )
    m1 = jnp.mean(x1, axis=-1, keepdims=True)
    v1 = jnp.mean((x1 - m1) * (x1 - m1), axis=-1, keepdims=True)
    h1 = (x1 - m1) / jnp.sqrt(v1 + 1e-5) * g1_ref[...] + be1_ref[...]
    h1 = jnp.maximum(h1, 0.0)

    x2 = jnp.dot(h1, w2_ref[...], precision=hp,
                 preferred_element_type=jnp.float32) + b2_ref[...]
    m2 = jnp.mean(x2, axis=-1, keepdims=True)
    v2 = jnp.mean((x2 - m2) * (x2 - m2), axis=-1, keepdims=True)
    h2 = (x2 - m2) / jnp.sqrt(v2 + 1e-5) * g2_ref[...] + be2_ref[...]
    h2 = jnp.maximum(h2, 0.0)

    out_ref[...] = jnp.dot(h2, w3_ref[...], precision=hp,
                           preferred_element_type=jnp.float32) + b3_ref[...]


@jax.jit
def _mlp(x_num, emb, W1a, W1b, b1, g1, be1, W2, b2, g2, be2, W3, b3):
    full = lambda shape: pl.BlockSpec(shape, lambda i: (0, 0))
    return pl.pallas_call(
        _mlp_body,
        grid=(B // BB,),
        in_specs=[
            pl.BlockSpec((BB, NUM_FEATURES), lambda i: (i, 0)),
            pl.BlockSpec((BB, F * D), lambda i: (i, 0)),
            full((NUM_FEATURES, H1)),
            full((F * D, H1)),
            full((1, H1)),
            full((1, H1)),
            full((1, H1)),
            full((H1, H2)),
            full((1, H2)),
            full((1, H2)),
            full((1, H2)),
            full((H2, 1)),
            full((1, 1)),
        ],
        out_specs=pl.BlockSpec((BB, 1), lambda i: (i, 0)),
        out_shape=jax.ShapeDtypeStruct((B, 1), jnp.float32),
    )(x_num, emb, W1a, W1b, b1.reshape(1, H1), g1.reshape(1, H1),
      be1.reshape(1, H1), W2, b2.reshape(1, H2), g2.reshape(1, H2),
      be2.reshape(1, H2), W3, b3.reshape(1, 1))


def kernel(x_num, x_cat, tables, W1, b1, g1, be1, W2, b2, g2, be2, W3, b3):
    idxU = (
        jnp.pad(x_cat.T, ((0, FP - F), (0, 0)))
        .reshape(FP // 8, 8, B // 128, 128)
        .transpose(0, 2, 1, 3)
        .reshape(FP // 8 * (B // 128) * 8, 128)
    )
    idx_rows = _sc_idx(idxU).reshape(N // CHUNK, CHUNK)
    emb = _sc_gather(tables, idx_rows).reshape(B, F * D)
    W1a = W1[:NUM_FEATURES]
    W1b = W1[NUM_FEATURES:]
    return _mlp(x_num, emb, W1a, W1b, b1, g1, be1, W2, b2, g2, be2, W3, b3)


# speedup vs baseline: 1.0123x; 1.0022x over previous
import functools

import jax
import jax.numpy as jnp
from jax import lax
from jax.experimental import pallas as pl
from jax.experimental.pallas import tpu as pltpu
from jax.experimental.pallas import tpu_sc as plsc

V = 100000
F = 26
D = 16
NUM_FEATURES = 13
B = 16384
H1 = 128
H2 = 64

N = B * F
NC = 2
NS = 16
NW = NC * NS
CHUNK = 128
ROWS = N // CHUNK
NCH = ROWS // NW
K = 8


def _sc_gather_body(tbl_hbm, idx_hbm, out_hbm, idx_v, buf, sem):
    c = lax.axis_index("c")
    s = lax.axis_index("s")
    wid = s * NC + c
    row0 = wid * NCH
    pltpu.sync_copy(idx_hbm.at[pl.ds(row0, NCH)], idx_v)

    def group(t, carry):
        j0 = t * K
        for b in range(K):
            pltpu.async_copy(
                tbl_hbm.at[idx_v.at[j0 + b]],
                buf.at[pl.ds(b * CHUNK, CHUNK)],
                sem,
            )
        for b in range(K):
            pltpu.make_async_copy(
                tbl_hbm.at[idx_v.at[j0 + b]],
                buf.at[pl.ds(b * CHUNK, CHUNK)],
                sem,
            ).wait()
        pltpu.sync_copy(
            buf, out_hbm.at[pl.ds((row0 + j0) * CHUNK, K * CHUNK)]
        )
        return carry

    lax.fori_loop(0, NCH // K, group, 0)


@jax.jit
def _sc_gather(tables, idx_rows):
    mesh = plsc.VectorSubcoreMesh(core_axis_name="c", subcore_axis_name="s")
    return pl.kernel(
        _sc_gather_body,
        out_type=jax.ShapeDtypeStruct((N, D), jnp.float32),
        mesh=mesh,
        compiler_params=pltpu.CompilerParams(use_tc_tiling_on_sc=False),
        scratch_types=[
            pltpu.VMEM((NCH, CHUNK), jnp.int32),
            pltpu.VMEM((K * CHUNK, D), jnp.float32),
            pltpu.SemaphoreType.DMA,
        ],
    )(tables, idx_rows)


BB = 1024


def _mlp_body(xn_ref, emb_ref, w1a_ref, w1b_ref, b1_ref, g1_ref, be1_ref,
              w2_ref, b2_ref, g2_ref, be2_ref, w3_ref, b3_ref, out_ref):
    hp = jax.lax.Precision.HIGHEST
    x1 = (
        jnp.dot(xn_ref[...], w1a_ref[...], precision=hp,
                preferred_element_type=jnp.float32)
        + jnp.dot(emb_ref[...], w1b_ref[...], precision=hp,
                  preferred_element_type=jnp.float32)
        + b1_ref[...]
    )
    m1 = jnp.mean(x1, axis=-1, keepdims=True)
    v1 = jnp.mean((x1 - m1) * (x1 - m1), axis=-1, keepdims=True)
    h1 = (x1 - m1) / jnp.sqrt(v1 + 1e-5) * g1_ref[...] + be1_ref[...]
    h1 = jnp.maximum(h1, 0.0)

    x2 = jnp.dot(h1, w2_ref[...], precision=hp,
                 preferred_element_type=jnp.float32) + b2_ref[...]
    m2 = jnp.mean(x2, axis=-1, keepdims=True)
    v2 = jnp.mean((x2 - m2) * (x2 - m2), axis=-1, keepdims=True)
    h2 = (x2 - m2) / jnp.sqrt(v2 + 1e-5) * g2_ref[...] + be2_ref[...]
    h2 = jnp.maximum(h2, 0.0)

    out_ref[...] = jnp.dot(h2, w3_ref[...], precision=hp,
                           preferred_element_type=jnp.float32) + b3_ref[...]


@jax.jit
def _mlp(x_num, emb, W1a, W1b, b1, g1, be1, W2, b2, g2, be2, W3, b3):
    full = lambda shape: pl.BlockSpec(shape, lambda i: (0, 0))
    return pl.pallas_call(
        _mlp_body,
        grid=(B // BB,),
        in_specs=[
            pl.BlockSpec((BB, NUM_FEATURES), lambda i: (i, 0)),
            pl.BlockSpec((BB, F * D), lambda i: (i, 0)),
            full((NUM_FEATURES, H1)),
            full((F * D, H1)),
            full((1, H1)),
            full((1, H1)),
            full((1, H1)),
            full((H1, H2)),
            full((1, H2)),
            full((1, H2)),
            full((1, H2)),
            full((H2, 1)),
            full((1, 1)),
        ],
        out_specs=pl.BlockSpec((BB, 1), lambda i: (i, 0)),
        out_shape=jax.ShapeDtypeStruct((B, 1), jnp.float32),
    )(x_num, emb, W1a, W1b, b1.reshape(1, H1), g1.reshape(1, H1),
      be1.reshape(1, H1), W2, b2.reshape(1, H2), g2.reshape(1, H2),
      be2.reshape(1, H2), W3, b3.reshape(1, 1))


def kernel(x_num, x_cat, tables, W1, b1, g1, be1, W2, b2, g2, be2, W3, b3):
    offs = (jnp.arange(F, dtype=jnp.int32) * V)[None, :]
    idx_rows = (x_cat + offs).reshape(ROWS, CHUNK)
    emb = _sc_gather(tables, idx_rows).reshape(B, F * D)
    W1a = W1[:NUM_FEATURES]
    W1b = W1[NUM_FEATURES:]
    return _mlp(x_num, emb, W1a, W1b, b1, g1, be1, W2, b2, g2, be2, W3, b3)
